# Initial kernel scaffold; baseline (speedup 1.0000x reference)
import jax
import jax.numpy as jnp
from jax.experimental import pallas as pl


def kernel(x, w1, b1, w2, b2, ws, bs):
    raise NotImplementedError("write your pallas kernel here")


# speedup vs baseline: 1.0348x; 1.0348x over previous
import functools

import jax
import jax.numpy as jnp
from jax.experimental import pallas as pl
from jax.experimental.pallas import tpu as pltpu

_EPS = 1e-5
_VMEM_LIMIT = 48 * 1024 * 1024


def _cparams():
    return pltpu.CompilerParams(
        dimension_semantics=("parallel",),
        vmem_limit_bytes=_VMEM_LIMIT,
    )


def _whole(shape):
    shape = tuple(shape)
    return pl.BlockSpec(shape, lambda n: (0,) * len(shape))


def _im2col(xp, H, W):
    C = xp.shape[-1]
    return jnp.concatenate(
        [xp[dy:dy + H, dx:dx + W, :].reshape(H * W, C)
         for dy in range(3) for dx in range(3)], axis=-1)


def _stats(y):
    s = jnp.sum(y, axis=0, keepdims=True)
    ss = jnp.sum(y * y, axis=0, keepdims=True)
    return jnp.concatenate([s, ss], axis=0)[None]


def _conv1_body(x_ref, w1_ref, b1_ref, ws_ref, bs_ref,
                y1_ref, sc_ref, st1_ref, sts_ref, xp, *, H, W):
    cin = x_ref.shape[-1]
    cout = w1_ref.shape[-1]

    xb = x_ref[0].astype(jnp.bfloat16)
    xp[...] = jnp.zeros((H + 2, W + 2, cin), jnp.bfloat16)
    xp[1:1 + H, 1:1 + W, :] = xb

    patches = _im2col(xp[...], H, W)
    y1 = jnp.dot(patches, w1_ref[...],
                 preferred_element_type=jnp.float32) + b1_ref[...]
    st1_ref[...] = _stats(y1)
    y1_ref[...] = y1.astype(jnp.bfloat16).reshape(1, H, W, cout)

    sc = jnp.dot(xb.reshape(H * W, cin), ws_ref[...],
                 preferred_element_type=jnp.float32) + bs_ref[...]
    sts_ref[...] = _stats(sc)
    sc_ref[...] = sc.astype(jnp.bfloat16).reshape(1, H, W * cout)


def _conv2_body(y1_ref, bn1_ref, w2_ref, b2_ref, y2_ref, st2_ref, hp,
                *, H, W):
    cout = w2_ref.shape[-1]
    bn1 = bn1_ref[...]
    h1 = jnp.maximum(y1_ref[0].astype(jnp.float32) * bn1[0] + bn1[1], 0.0)

    hp[...] = jnp.zeros((H + 2, W + 2, cout), jnp.bfloat16)
    hp[1:1 + H, 1:1 + W, :] = h1.astype(jnp.bfloat16)

    patches = _im2col(hp[...], H, W)
    y2 = jnp.dot(patches, w2_ref[...],
                 preferred_element_type=jnp.float32) + b2_ref[...]
    st2_ref[...] = _stats(y2)
    y2_ref[...] = y2.astype(jnp.bfloat16).reshape(1, H, W * cout)


def _add_relu_body(y2_ref, sc_ref, bn2_ref, bns_ref, o_ref):
    bn2 = bn2_ref[...]
    bns = bns_ref[...]
    sc = sc_ref[...].astype(jnp.float32) * bns[0] + bns[1]
    y2 = y2_ref[...].astype(jnp.float32) * bn2[0] + bn2[1]
    o_ref[...] = jnp.maximum(y2 + sc, 0.0)


def _bn_scale_shift(stats, count):
    tot = jnp.sum(stats, axis=0)
    mean = tot[0] / count
    var = jnp.maximum(tot[1] / count - mean * mean, 0.0)
    inv = jax.lax.rsqrt(var + _EPS)
    return inv, -mean * inv


def kernel(x, w1, b1, w2, b2, ws, bs):
    xh = jnp.transpose(x, (0, 2, 3, 1)).astype(jnp.float32)
    N, H, W, cin = xh.shape
    cout = w1.shape[-1]
    count = N * H * W

    w1f = w1.reshape(9 * cin, cout).astype(jnp.bfloat16)
    w2f = w2.reshape(9 * cout, cout).astype(jnp.bfloat16)
    wsf = ws.astype(jnp.bfloat16)
    b1f = b1.reshape(1, cout)
    b2f = b2.reshape(1, cout)
    bsf = bs.reshape(1, cout)

    grid = (N,)
    x_spec = pl.BlockSpec((1, H, W, cin), lambda n: (n, 0, 0, 0))
    y1_spec = pl.BlockSpec((1, H, W, cout), lambda n: (n, 0, 0, 0))
    dense_spec = pl.BlockSpec((1, H, W * cout), lambda n: (n, 0, 0))
    st_spec = pl.BlockSpec((1, 2, cout), lambda n: (n, 0, 0))

    y1, sc, st1, sts = pl.pallas_call(
        functools.partial(_conv1_body, H=H, W=W),
        grid=grid,
        in_specs=[x_spec, _whole(w1f.shape), _whole((1, cout)),
                  _whole(wsf.shape), _whole((1, cout))],
        out_specs=(y1_spec, dense_spec, st_spec, st_spec),
        out_shape=(
            jax.ShapeDtypeStruct((N, H, W, cout), jnp.bfloat16),
            jax.ShapeDtypeStruct((N, H, W * cout), jnp.bfloat16),
            jax.ShapeDtypeStruct((N, 2, cout), jnp.float32),
            jax.ShapeDtypeStruct((N, 2, cout), jnp.float32),
        ),
        scratch_shapes=[pltpu.VMEM((H + 2, W + 2, cin), jnp.bfloat16)],
        compiler_params=_cparams(),
        cost_estimate=pl.CostEstimate(
            flops=2 * count * (9 * cin + cin) * cout, transcendentals=0,
            bytes_accessed=4 * N * H * W * cin + 2 * 2 * N * H * W * cout),
    )(xh, w1f, b1f, wsf, bsf)

    s1, t1 = _bn_scale_shift(st1, count)
    bn1 = jnp.stack([s1, t1])

    y2, st2 = pl.pallas_call(
        functools.partial(_conv2_body, H=H, W=W),
        grid=grid,
        in_specs=[y1_spec, _whole((2, cout)), _whole(w2f.shape),
                  _whole((1, cout))],
        out_specs=(dense_spec, st_spec),
        out_shape=(jax.ShapeDtypeStruct((N, H, W * cout), jnp.bfloat16),
                   jax.ShapeDtypeStruct((N, 2, cout), jnp.float32)),
        scratch_shapes=[pltpu.VMEM((H + 2, W + 2, cout), jnp.bfloat16)],
        compiler_params=_cparams(),
        cost_estimate=pl.CostEstimate(
            flops=2 * count * 9 * cout * cout, transcendentals=0,
            bytes_accessed=2 * 2 * N * H * W * cout),
    )(y1, bn1, w2f, b2f)

    s2, t2 = _bn_scale_shift(st2, count)
    bn2_lane = jnp.stack([jnp.tile(s2, W), jnp.tile(t2, W)])
    ss_, ts_ = _bn_scale_shift(sts, count)
    bns_lane = jnp.stack([jnp.tile(ss_, W), jnp.tile(ts_, W)])

    out = pl.pallas_call(
        _add_relu_body,
        grid=grid,
        in_specs=[dense_spec, dense_spec,
                  _whole((2, W * cout)), _whole((2, W * cout))],
        out_specs=dense_spec,
        out_shape=jax.ShapeDtypeStruct((N, H, W * cout), jnp.float32),
        compiler_params=_cparams(),
        cost_estimate=pl.CostEstimate(
            flops=6 * count * cout, transcendentals=0,
            bytes_accessed=2 * 2 * N * H * W * cout + 4 * N * H * W * cout),
    )(y2, sc, bn2_lane, bns_lane)

    out = out.reshape(N, H, W, cout)
    return jnp.transpose(out, (0, 3, 1, 2))


# speedup vs baseline: 1.0662x; 1.0304x over previous
import functools

import jax
import jax.numpy as jnp
from jax.experimental import pallas as pl
from jax.experimental.pallas import tpu as pltpu

_EPS = 1e-5
_VMEM_LIMIT = 48 * 1024 * 1024


def _cparams():
    return pltpu.CompilerParams(
        dimension_semantics=("parallel",),
        vmem_limit_bytes=_VMEM_LIMIT,
    )


def _whole(shape):
    shape = tuple(shape)
    return pl.BlockSpec(shape, lambda n: (0,) * len(shape))


def _im2col(xp, H, W):
    C = xp.shape[-1]
    return jnp.concatenate(
        [xp[dy:dy + H, dx:dx + W, :].reshape(H * W, C)
         for dy in range(3) for dx in range(3)], axis=-1)


def _row_stats(y):
    s = jnp.sum(y, axis=0, keepdims=True)
    ss = jnp.sum(y * y, axis=0, keepdims=True)
    return jnp.concatenate([s, ss], axis=0)[None]


def _col_stats(y):
    s = jnp.sum(y, axis=1, keepdims=True)
    ss = jnp.sum(y * y, axis=1, keepdims=True)
    return jnp.concatenate([s, ss], axis=1)[None]


def _conv1_body(x_ref, w1_ref, b1_ref, ws_ref, bs_ref,
                y1_ref, sct_ref, st1_ref, sts_ref, xp, *, H, W):
    cin = x_ref.shape[1]
    cout = w1_ref.shape[-1]

    xcm = x_ref[0].astype(jnp.bfloat16)
    xt = jnp.transpose(xcm).reshape(H, W, cin)

    xp[...] = jnp.zeros((H + 2, W + 2, cin), jnp.bfloat16)
    xp[1:1 + H, 1:1 + W, :] = xt

    patches = _im2col(xp[...], H, W)
    y1 = jnp.dot(patches, w1_ref[...],
                 preferred_element_type=jnp.float32) + b1_ref[...]
    st1_ref[...] = _row_stats(y1)
    y1_ref[...] = y1.astype(jnp.bfloat16)[None]

    sct = jax.lax.dot_general(
        ws_ref[...], xcm, (((0,), (0,)), ((), ())),
        preferred_element_type=jnp.float32) + bs_ref[...]
    sts_ref[...] = _col_stats(sct)
    sct_ref[...] = sct.astype(jnp.bfloat16)[None]


def _conv2_body(y1_ref, bn1_ref, w2_ref, b2_ref, y2t_ref, st2_ref, hp,
                *, H, W):
    cout = w2_ref.shape[-1]
    bn1 = bn1_ref[...]
    h1 = jnp.maximum(y1_ref[0].astype(jnp.float32) * bn1[0] + bn1[1], 0.0)

    hp[...] = jnp.zeros((H + 2, W + 2, cout), jnp.bfloat16)
    hp[1:1 + H, 1:1 + W, :] = h1.astype(jnp.bfloat16).reshape(H, W, cout)

    patches = _im2col(hp[...], H, W)
    y2 = jnp.dot(patches, w2_ref[...],
                 preferred_element_type=jnp.float32) + b2_ref[...]
    st2_ref[...] = _row_stats(y2)
    y2t_ref[...] = jnp.transpose(y2.astype(jnp.bfloat16))[None]


def _add_relu_body(y2t_ref, sct_ref, bnc_ref, o_ref):
    bnc = bnc_ref[...]
    y2 = y2t_ref[0].astype(jnp.float32) * bnc[:, 0:1] + bnc[:, 1:2]
    sc = sct_ref[0].astype(jnp.float32) * bnc[:, 2:3] + bnc[:, 3:4]
    o_ref[...] = jnp.maximum(y2 + sc, 0.0)[None]


def _scale_shift(tot_sum, tot_sq, count):
    mean = tot_sum / count
    var = jnp.maximum(tot_sq / count - mean * mean, 0.0)
    inv = jax.lax.rsqrt(var + _EPS)
    return inv, -mean * inv


def kernel(x, w1, b1, w2, b2, ws, bs):
    N, cin, H, W = x.shape
    cout = w1.shape[-1]
    HW = H * W
    count = N * HW
    xf = x.astype(jnp.float32).reshape(N, cin, HW)

    w1f = w1.reshape(9 * cin, cout).astype(jnp.bfloat16)
    w2f = w2.reshape(9 * cout, cout).astype(jnp.bfloat16)
    wsf = ws.astype(jnp.bfloat16)
    b1f = b1.reshape(1, cout)
    b2f = b2.reshape(1, cout)
    bsf = bs.reshape(cout, 1)

    grid = (N,)
    xcm_spec = pl.BlockSpec((1, cin, HW), lambda n: (n, 0, 0))
    y1_spec = pl.BlockSpec((1, HW, cout), lambda n: (n, 0, 0))
    cm_spec = pl.BlockSpec((1, cout, HW), lambda n: (n, 0, 0))
    st_spec = pl.BlockSpec((1, 2, cout), lambda n: (n, 0, 0))
    stc_spec = pl.BlockSpec((1, cout, 2), lambda n: (n, 0, 0))

    y1, sct, st1, sts = pl.pallas_call(
        functools.partial(_conv1_body, H=H, W=W),
        grid=grid,
        in_specs=[xcm_spec, _whole(w1f.shape), _whole((1, cout)),
                  _whole(wsf.shape), _whole((cout, 1))],
        out_specs=(y1_spec, cm_spec, st_spec, stc_spec),
        out_shape=(
            jax.ShapeDtypeStruct((N, HW, cout), jnp.bfloat16),
            jax.ShapeDtypeStruct((N, cout, HW), jnp.bfloat16),
            jax.ShapeDtypeStruct((N, 2, cout), jnp.float32),
            jax.ShapeDtypeStruct((N, cout, 2), jnp.float32),
        ),
        scratch_shapes=[pltpu.VMEM((H + 2, W + 2, cin), jnp.bfloat16)],
        compiler_params=_cparams(),
        cost_estimate=pl.CostEstimate(
            flops=2 * count * (9 * cin + cin) * cout, transcendentals=0,
            bytes_accessed=4 * count * cin + 2 * 2 * count * cout),
    )(xf, w1f, b1f, wsf, bsf)

    s1, t1 = _scale_shift(jnp.sum(st1[:, 0], 0), jnp.sum(st1[:, 1], 0), count)
    bn1 = jnp.stack([s1, t1])

    y2t, st2 = pl.pallas_call(
        functools.partial(_conv2_body, H=H, W=W),
        grid=grid,
        in_specs=[y1_spec, _whole((2, cout)), _whole(w2f.shape),
                  _whole((1, cout))],
        out_specs=(cm_spec, st_spec),
        out_shape=(jax.ShapeDtypeStruct((N, cout, HW), jnp.bfloat16),
                   jax.ShapeDtypeStruct((N, 2, cout), jnp.float32)),
        scratch_shapes=[pltpu.VMEM((H + 2, W + 2, cout), jnp.bfloat16)],
        compiler_params=_cparams(),
        cost_estimate=pl.CostEstimate(
            flops=2 * count * 9 * cout * cout, transcendentals=0,
            bytes_accessed=2 * 2 * count * cout),
    )(y1, bn1, w2f, b2f)

    s2, t2 = _scale_shift(jnp.sum(st2[:, 0], 0), jnp.sum(st2[:, 1], 0), count)
    ss_, ts_ = _scale_shift(jnp.sum(sts[:, :, 0], 0), jnp.sum(sts[:, :, 1], 0),
                            count)
    bnc = jnp.stack([s2, t2, ss_, ts_], axis=1)

    out = pl.pallas_call(
        _add_relu_body,
        grid=grid,
        in_specs=[cm_spec, cm_spec, _whole((cout, 4))],
        out_specs=cm_spec,
        out_shape=jax.ShapeDtypeStruct((N, cout, HW), jnp.float32),
        compiler_params=_cparams(),
        cost_estimate=pl.CostEstimate(
            flops=6 * count * cout, transcendentals=0,
            bytes_accessed=2 * 2 * count * cout + 4 * count * cout),
    )(y2t, sct, bnc)

    return out.reshape(N, cout, H, W)


# speedup vs baseline: 1.2104x; 1.1352x over previous
import functools

import jax
import jax.numpy as jnp
from jax.experimental import pallas as pl
from jax.experimental.pallas import tpu as pltpu

_EPS = 1e-5
_VMEM_LIMIT = 64 * 1024 * 1024


def _cparams():
    return pltpu.CompilerParams(
        dimension_semantics=("parallel",),
        vmem_limit_bytes=_VMEM_LIMIT,
    )


def _whole(shape):
    shape = tuple(shape)
    return pl.BlockSpec(shape, lambda n: (0,) * len(shape))


def _im2col(xp, H, W):
    C = xp.shape[-1]
    return jnp.concatenate(
        [xp[dy:dy + H, dx:dx + W, :].reshape(H * W, C)
         for dy in range(3) for dx in range(3)], axis=-1)


def _conv1_body(x_ref, w1_ref, b1_ref, ws_ref, bs_ref,
                y1_ref, sct_ref, st_ref, xp, *, H, W, imgs):
    cin = x_ref.shape[1]
    cout = w1_ref.shape[-1]
    HW = H * W

    s1 = jnp.zeros((1, cout), jnp.float32)
    q1 = jnp.zeros((1, cout), jnp.float32)
    ssc = jnp.zeros((cout, 1), jnp.float32)
    qsc = jnp.zeros((cout, 1), jnp.float32)
    for i in range(imgs):
        xcm = x_ref[i].astype(jnp.bfloat16)
        xt = jnp.transpose(xcm).reshape(H, W, cin)

        xp[...] = jnp.zeros((H + 2, W + 2, cin), jnp.bfloat16)
        xp[1:1 + H, 1:1 + W, :] = xt

        patches = _im2col(xp[...], H, W)
        y1 = jnp.dot(patches, w1_ref[...],
                     preferred_element_type=jnp.float32) + b1_ref[...]
        s1 = s1 + jnp.sum(y1, axis=0, keepdims=True)
        q1 = q1 + jnp.sum(y1 * y1, axis=0, keepdims=True)
        y1_ref[i] = y1.astype(jnp.bfloat16)

        sct = jax.lax.dot_general(
            ws_ref[...], xcm, (((0,), (0,)), ((), ())),
            preferred_element_type=jnp.float32) + bs_ref[...]
        ssc = ssc + jnp.sum(sct, axis=1, keepdims=True)
        qsc = qsc + jnp.sum(sct * sct, axis=1, keepdims=True)
        sct_ref[i] = sct.astype(jnp.bfloat16)

    sc_rows = jnp.transpose(jnp.concatenate([ssc, qsc], axis=1))
    st_ref[...] = jnp.concatenate([s1, q1, sc_rows], axis=0)[None]


def _conv2_body(y1_ref, bn1_ref, w2_ref, b2_ref, y2t_ref, st_ref, hp,
                *, H, W, imgs):
    cout = w2_ref.shape[-1]
    bn1 = bn1_ref[...]

    s2 = jnp.zeros((1, cout), jnp.float32)
    q2 = jnp.zeros((1, cout), jnp.float32)
    for i in range(imgs):
        h1 = jnp.maximum(y1_ref[i].astype(jnp.float32) * bn1[0] + bn1[1], 0.0)

        hp[...] = jnp.zeros((H + 2, W + 2, cout), jnp.bfloat16)
        hp[1:1 + H, 1:1 + W, :] = h1.astype(jnp.bfloat16).reshape(H, W, cout)

        patches = _im2col(hp[...], H, W)
        y2 = jnp.dot(patches, w2_ref[...],
                     preferred_element_type=jnp.float32) + b2_ref[...]
        s2 = s2 + jnp.sum(y2, axis=0, keepdims=True)
        q2 = q2 + jnp.sum(y2 * y2, axis=0, keepdims=True)
        y2t_ref[i] = jnp.transpose(y2.astype(jnp.bfloat16))

    st_ref[...] = jnp.concatenate([s2, q2], axis=0)[None]


def _add_relu_body(y2t_ref, sct_ref, bnc_ref, o_ref):
    bnc = bnc_ref[...]
    s2 = bnc[:, 0:1][None]
    t2 = bnc[:, 1:2][None]
    ss = bnc[:, 2:3][None]
    ts = bnc[:, 3:4][None]
    y2 = y2t_ref[...].astype(jnp.float32) * s2 + t2
    sc = sct_ref[...].astype(jnp.float32) * ss + ts
    o_ref[...] = jnp.maximum(y2 + sc, 0.0)


def _scale_shift(tot_sum, tot_sq, count):
    mean = tot_sum / count
    var = jnp.maximum(tot_sq / count - mean * mean, 0.0)
    inv = jax.lax.rsqrt(var + _EPS)
    return inv, -mean * inv


def kernel(x, w1, b1, w2, b2, ws, bs):
    N, cin, H, W = x.shape
    cout = w1.shape[-1]
    HW = H * W
    count = N * HW
    imgs = 4 if N % 4 == 0 else (2 if N % 2 == 0 else 1)
    G = N // imgs
    xf = x.astype(jnp.float32).reshape(N, cin, HW)

    w1f = w1.reshape(9 * cin, cout).astype(jnp.bfloat16)
    w2f = w2.reshape(9 * cout, cout).astype(jnp.bfloat16)
    wsf = ws.astype(jnp.bfloat16)
    b1f = b1.reshape(1, cout)
    b2f = b2.reshape(1, cout)
    bsf = bs.reshape(cout, 1)

    grid = (G,)
    xcm_spec = pl.BlockSpec((imgs, cin, HW), lambda n: (n, 0, 0))
    y1_spec = pl.BlockSpec((imgs, HW, cout), lambda n: (n, 0, 0))
    cm_spec = pl.BlockSpec((imgs, cout, HW), lambda n: (n, 0, 0))
    st4_spec = pl.BlockSpec((1, 4, cout), lambda n: (n, 0, 0))
    st2_spec = pl.BlockSpec((1, 2, cout), lambda n: (n, 0, 0))

    y1, sct, sta = pl.pallas_call(
        functools.partial(_conv1_body, H=H, W=W, imgs=imgs),
        grid=grid,
        in_specs=[xcm_spec, _whole(w1f.shape), _whole((1, cout)),
                  _whole(wsf.shape), _whole((cout, 1))],
        out_specs=(y1_spec, cm_spec, st4_spec),
        out_shape=(
            jax.ShapeDtypeStruct((N, HW, cout), jnp.bfloat16),
            jax.ShapeDtypeStruct((N, cout, HW), jnp.bfloat16),
            jax.ShapeDtypeStruct((G, 4, cout), jnp.float32),
        ),
        scratch_shapes=[pltpu.VMEM((H + 2, W + 2, cin), jnp.bfloat16)],
        compiler_params=_cparams(),
        cost_estimate=pl.CostEstimate(
            flops=2 * count * (9 * cin + cin) * cout, transcendentals=0,
            bytes_accessed=4 * count * cin + 2 * 2 * count * cout),
    )(xf, w1f, b1f, wsf, bsf)

    s1, t1 = _scale_shift(jnp.sum(sta[:, 0], 0), jnp.sum(sta[:, 1], 0), count)
    bn1 = jnp.stack([s1, t1])

    y2t, stb = pl.pallas_call(
        functools.partial(_conv2_body, H=H, W=W, imgs=imgs),
        grid=grid,
        in_specs=[y1_spec, _whole((2, cout)), _whole(w2f.shape),
                  _whole((1, cout))],
        out_specs=(cm_spec, st2_spec),
        out_shape=(jax.ShapeDtypeStruct((N, cout, HW), jnp.bfloat16),
                   jax.ShapeDtypeStruct((G, 2, cout), jnp.float32)),
        scratch_shapes=[pltpu.VMEM((H + 2, W + 2, cout), jnp.bfloat16)],
        compiler_params=_cparams(),
        cost_estimate=pl.CostEstimate(
            flops=2 * count * 9 * cout * cout, transcendentals=0,
            bytes_accessed=2 * 2 * count * cout),
    )(y1, bn1, w2f, b2f)

    s2, t2 = _scale_shift(jnp.sum(stb[:, 0], 0), jnp.sum(stb[:, 1], 0), count)
    ss_, ts_ = _scale_shift(jnp.sum(sta[:, 2], 0), jnp.sum(sta[:, 3], 0),
                            count)
    bnc = jnp.stack([s2, t2, ss_, ts_], axis=1)

    out = pl.pallas_call(
        _add_relu_body,
        grid=grid,
        in_specs=[cm_spec, cm_spec, _whole((cout, 4))],
        out_specs=cm_spec,
        out_shape=jax.ShapeDtypeStruct((N, cout, HW), jnp.float32),
        compiler_params=_cparams(),
        cost_estimate=pl.CostEstimate(
            flops=6 * count * cout, transcendentals=0,
            bytes_accessed=2 * 2 * count * cout + 4 * count * cout),
    )(y2t, sct, bnc)

    return out.reshape(N, cout, H, W)


# speedup vs baseline: 1.2354x; 1.0206x over previous
import functools

import jax
import jax.numpy as jnp
from jax.experimental import pallas as pl
from jax.experimental.pallas import tpu as pltpu

_EPS = 1e-5
_VMEM_LIMIT = 64 * 1024 * 1024


def _cparams():
    return pltpu.CompilerParams(
        dimension_semantics=("parallel",),
        vmem_limit_bytes=_VMEM_LIMIT,
    )


def _whole(shape):
    shape = tuple(shape)
    return pl.BlockSpec(shape, lambda n: (0,) * len(shape))


def _im2col(xp, H, W):
    C = xp.shape[-1]
    return jnp.concatenate(
        [xp[dy:dy + H, dx:dx + W, :].reshape(H * W, C)
         for dy in range(3) for dx in range(3)], axis=-1)


def _scale_shift_rows(sum_row, sq_row, count):
    mean = sum_row / count
    var = jnp.maximum(sq_row / count - mean * mean, 0.0)
    inv = jax.lax.rsqrt(var + _EPS)
    return inv, -mean * inv


def _conv1_body(x_ref, w1_ref, b1_ref, ws_ref, bs_ref,
                y1_ref, sct_ref, st_ref, xp, *, H, W, imgs):
    cin = x_ref.shape[1]
    cout = w1_ref.shape[-1]

    s1 = jnp.zeros((1, cout), jnp.float32)
    q1 = jnp.zeros((1, cout), jnp.float32)
    ssc = jnp.zeros((cout, 1), jnp.float32)
    qsc = jnp.zeros((cout, 1), jnp.float32)
    for i in range(imgs):
        xcm = x_ref[i].astype(jnp.bfloat16)
        xt = jnp.transpose(xcm).reshape(H, W, cin)

        xp[...] = jnp.zeros((H + 2, W + 2, cin), jnp.bfloat16)
        xp[1:1 + H, 1:1 + W, :] = xt

        patches = _im2col(xp[...], H, W)
        y1 = jnp.dot(patches, w1_ref[...],
                     preferred_element_type=jnp.float32) + b1_ref[...]
        s1 = s1 + jnp.sum(y1, axis=0, keepdims=True)
        q1 = q1 + jnp.sum(y1 * y1, axis=0, keepdims=True)
        y1_ref[i] = y1.astype(jnp.bfloat16)

        sct = jax.lax.dot_general(
            ws_ref[...], xcm, (((0,), (0,)), ((), ())),
            preferred_element_type=jnp.float32) + bs_ref[...]
        ssc = ssc + jnp.sum(sct, axis=1, keepdims=True)
        qsc = qsc + jnp.sum(sct * sct, axis=1, keepdims=True)
        sct_ref[i] = sct.astype(jnp.bfloat16)

    sc_rows = jnp.transpose(jnp.concatenate([ssc, qsc], axis=1))
    st_ref[...] = jnp.concatenate([s1, q1, sc_rows], axis=0)[None]


def _conv2_body(y1_ref, sta_ref, w2_ref, b2_ref, y2t_ref, st_ref, hp,
                *, H, W, imgs, count):
    cout = w2_ref.shape[-1]
    tot = jnp.sum(sta_ref[...], axis=0)
    scale, shift = _scale_shift_rows(tot[0:1], tot[1:2], count)

    s2 = jnp.zeros((1, cout), jnp.float32)
    q2 = jnp.zeros((1, cout), jnp.float32)
    for i in range(imgs):
        h1 = jnp.maximum(y1_ref[i].astype(jnp.float32) * scale + shift, 0.0)

        hp[...] = jnp.zeros((H + 2, W + 2, cout), jnp.bfloat16)
        hp[1:1 + H, 1:1 + W, :] = h1.astype(jnp.bfloat16).reshape(H, W, cout)

        patches = _im2col(hp[...], H, W)
        y2 = jnp.dot(patches, w2_ref[...],
                     preferred_element_type=jnp.float32) + b2_ref[...]
        s2 = s2 + jnp.sum(y2, axis=0, keepdims=True)
        q2 = q2 + jnp.sum(y2 * y2, axis=0, keepdims=True)
        y2t_ref[i] = jnp.transpose(y2.astype(jnp.bfloat16))

    st_ref[...] = jnp.concatenate([s2, q2], axis=0)[None]


def _add_relu_body(y2t_ref, sct_ref, sta_ref, stb_ref, o_ref, *, count):
    tota = jnp.sum(sta_ref[...], axis=0)
    totb = jnp.sum(stb_ref[...], axis=0)
    s2r, t2r = _scale_shift_rows(totb[0:1], totb[1:2], count)
    ssr, tsr = _scale_shift_rows(tota[2:3], tota[3:4], count)
    bnc = jnp.transpose(jnp.concatenate([s2r, t2r, ssr, tsr], axis=0))
    s2 = bnc[:, 0:1][None]
    t2 = bnc[:, 1:2][None]
    ss = bnc[:, 2:3][None]
    ts = bnc[:, 3:4][None]
    y2 = y2t_ref[...].astype(jnp.float32) * s2 + t2
    sc = sct_ref[...].astype(jnp.float32) * ss + ts
    o_ref[...] = jnp.maximum(y2 + sc, 0.0)


def kernel(x, w1, b1, w2, b2, ws, bs):
    N, cin, H, W = x.shape
    cout = w1.shape[-1]
    HW = H * W
    count = N * HW
    imgs = 8 if N % 8 == 0 else (2 if N % 2 == 0 else 1)
    imgs_c = 4 if N % 4 == 0 else 1
    G = N // imgs
    Gc = N // imgs_c
    xf = x.astype(jnp.float32).reshape(N, cin, HW)

    w1f = w1.reshape(9 * cin, cout).astype(jnp.bfloat16)
    w2f = w2.reshape(9 * cout, cout).astype(jnp.bfloat16)
    wsf = ws.astype(jnp.bfloat16)
    b1f = b1.reshape(1, cout)
    b2f = b2.reshape(1, cout)
    bsf = bs.reshape(cout, 1)

    xcm_spec = pl.BlockSpec((imgs, cin, HW), lambda n: (n, 0, 0))
    y1_spec = pl.BlockSpec((imgs, HW, cout), lambda n: (n, 0, 0))
    cm_spec = pl.BlockSpec((imgs, cout, HW), lambda n: (n, 0, 0))
    cmc_spec = pl.BlockSpec((imgs_c, cout, HW), lambda n: (n, 0, 0))
    st4_spec = pl.BlockSpec((1, 4, cout), lambda n: (n, 0, 0))
    st2_spec = pl.BlockSpec((1, 2, cout), lambda n: (n, 0, 0))

    y1, sct, sta = pl.pallas_call(
        functools.partial(_conv1_body, H=H, W=W, imgs=imgs),
        grid=(G,),
        in_specs=[xcm_spec, _whole(w1f.shape), _whole((1, cout)),
                  _whole(wsf.shape), _whole((cout, 1))],
        out_specs=(y1_spec, cm_spec, st4_spec),
        out_shape=(
            jax.ShapeDtypeStruct((N, HW, cout), jnp.bfloat16),
            jax.ShapeDtypeStruct((N, cout, HW), jnp.bfloat16),
            jax.ShapeDtypeStruct((G, 4, cout), jnp.float32),
        ),
        scratch_shapes=[pltpu.VMEM((H + 2, W + 2, cin), jnp.bfloat16)],
        compiler_params=_cparams(),
        cost_estimate=pl.CostEstimate(
            flops=2 * count * (9 * cin + cin) * cout, transcendentals=0,
            bytes_accessed=4 * count * cin + 2 * 2 * count * cout),
    )(xf, w1f, b1f, wsf, bsf)

    y2t, stb = pl.pallas_call(
        functools.partial(_conv2_body, H=H, W=W, imgs=imgs, count=count),
        grid=(G,),
        in_specs=[y1_spec, _whole((G, 4, cout)), _whole(w2f.shape),
                  _whole((1, cout))],
        out_specs=(cm_spec, st2_spec),
        out_shape=(jax.ShapeDtypeStruct((N, cout, HW), jnp.bfloat16),
                   jax.ShapeDtypeStruct((G, 2, cout), jnp.float32)),
        scratch_shapes=[pltpu.VMEM((H + 2, W + 2, cout), jnp.bfloat16)],
        compiler_params=_cparams(),
        cost_estimate=pl.CostEstimate(
            flops=2 * count * 9 * cout * cout, transcendentals=0,
            bytes_accessed=2 * 2 * count * cout),
    )(y1, sta, w2f, b2f)

    out = pl.pallas_call(
        functools.partial(_add_relu_body, count=count),
        grid=(Gc,),
        in_specs=[cmc_spec, cmc_spec, _whole((G, 4, cout)),
                  _whole((G, 2, cout))],
        out_specs=cmc_spec,
        out_shape=jax.ShapeDtypeStruct((N, cout, HW), jnp.float32),
        compiler_params=_cparams(),
        cost_estimate=pl.CostEstimate(
            flops=6 * count * cout, transcendentals=0,
            bytes_accessed=2 * 2 * count * cout + 4 * count * cout),
    )(y2t, sct, sta, stb)

    return out.reshape(N, cout, H, W)


# speedup vs baseline: 1.2361x; 1.0006x over previous
import functools

import jax
import jax.numpy as jnp
from jax.experimental import pallas as pl
from jax.experimental.pallas import tpu as pltpu

_EPS = 1e-5
_VMEM_LIMIT = 64 * 1024 * 1024


def _cparams():
    return pltpu.CompilerParams(
        dimension_semantics=("parallel",),
        vmem_limit_bytes=_VMEM_LIMIT,
    )


def _whole(shape):
    shape = tuple(shape)
    return pl.BlockSpec(shape, lambda n: (0,) * len(shape))


def _im2col(xp, H, W):
    C = xp.shape[-1]
    return jnp.concatenate(
        [xp[dy:dy + H, dx:dx + W, :].reshape(H * W, C)
         for dy in range(3) for dx in range(3)], axis=-1)


def _scale_shift_rows(sum_row, sq_row, count):
    mean = sum_row / count
    var = jnp.maximum(sq_row / count - mean * mean, 0.0)
    inv = jax.lax.rsqrt(var + _EPS)
    return inv, -mean * inv


def _conv1_body(x_ref, w1_ref, b1_ref, ws_ref, bs_ref,
                y1_ref, sct_ref, st_ref, xp, *, H, W, imgs):
    cin = x_ref.shape[1]
    cout = w1_ref.shape[-1]

    s1 = jnp.zeros((1, cout), jnp.float32)
    q1 = jnp.zeros((1, cout), jnp.float32)
    ssc = jnp.zeros((cout, 1), jnp.float32)
    qsc = jnp.zeros((cout, 1), jnp.float32)
    for i in range(imgs):
        xcm = x_ref[i].astype(jnp.bfloat16)
        xt = jnp.transpose(xcm).reshape(H, W, cin)

        xp[...] = jnp.zeros((H + 2, W + 2, cin), jnp.bfloat16)
        xp[1:1 + H, 1:1 + W, :] = xt

        patches = _im2col(xp[...], H, W)
        y1 = jnp.dot(patches, w1_ref[...],
                     preferred_element_type=jnp.float32) + b1_ref[...]
        s1 = s1 + jnp.sum(y1, axis=0, keepdims=True)
        q1 = q1 + jnp.sum(y1 * y1, axis=0, keepdims=True)
        y1_ref[i] = y1.astype(jnp.bfloat16)

        sct = jax.lax.dot_general(
            ws_ref[...], xcm, (((0,), (0,)), ((), ())),
            preferred_element_type=jnp.float32) + bs_ref[...]
        ssc = ssc + jnp.sum(sct, axis=1, keepdims=True)
        qsc = qsc + jnp.sum(sct * sct, axis=1, keepdims=True)
        sct_ref[i] = sct.astype(jnp.bfloat16)

    sc_rows = jnp.transpose(jnp.concatenate([ssc, qsc], axis=1))
    st_ref[...] = jnp.concatenate([s1, q1, sc_rows], axis=0)[None]


def _conv2_body(y1_ref, sta_ref, w2_ref, b2_ref, y2_ref, st_ref, hp,
                *, H, W, imgs, count):
    cout = w2_ref.shape[-1]
    tot = jnp.sum(sta_ref[...], axis=0)
    scale, shift = _scale_shift_rows(tot[0:1], tot[1:2], count)

    s2 = jnp.zeros((1, cout), jnp.float32)
    q2 = jnp.zeros((1, cout), jnp.float32)
    for i in range(imgs):
        h1 = jnp.maximum(y1_ref[i].astype(jnp.float32) * scale + shift, 0.0)

        hp[...] = jnp.zeros((H + 2, W + 2, cout), jnp.bfloat16)
        hp[1:1 + H, 1:1 + W, :] = h1.astype(jnp.bfloat16).reshape(H, W, cout)

        patches = _im2col(hp[...], H, W)
        y2 = jnp.dot(patches, w2_ref[...],
                     preferred_element_type=jnp.float32) + b2_ref[...]
        s2 = s2 + jnp.sum(y2, axis=0, keepdims=True)
        q2 = q2 + jnp.sum(y2 * y2, axis=0, keepdims=True)
        y2_ref[i] = y2.astype(jnp.bfloat16)

    st_ref[...] = jnp.concatenate([s2, q2], axis=0)[None]


def _add_relu_body(y2_ref, sct_ref, sta_ref, stb_ref, o_ref, *, count, imgs):
    tota = jnp.sum(sta_ref[...], axis=0)
    totb = jnp.sum(stb_ref[...], axis=0)
    s2r, t2r = _scale_shift_rows(totb[0:1], totb[1:2], count)
    ssr, tsr = _scale_shift_rows(tota[2:3], tota[3:4], count)
    bnc = jnp.transpose(jnp.concatenate([s2r, t2r, ssr, tsr], axis=0))
    s2 = bnc[:, 0:1]
    t2 = bnc[:, 1:2]
    ss = bnc[:, 2:3]
    ts = bnc[:, 3:4]
    for i in range(imgs):
        y2t = jnp.transpose(y2_ref[i]).astype(jnp.float32) * s2 + t2
        sc = sct_ref[i].astype(jnp.float32) * ss + ts
        o_ref[i] = jnp.maximum(y2t + sc, 0.0)


def kernel(x, w1, b1, w2, b2, ws, bs):
    N, cin, H, W = x.shape
    cout = w1.shape[-1]
    HW = H * W
    count = N * HW
    imgs = 8 if N % 8 == 0 else (2 if N % 2 == 0 else 1)
    imgs_c = 4 if N % 4 == 0 else 1
    G = N // imgs
    Gc = N // imgs_c
    xf = x.astype(jnp.float32).reshape(N, cin, HW)

    w1f = w1.reshape(9 * cin, cout).astype(jnp.bfloat16)
    w2f = w2.reshape(9 * cout, cout).astype(jnp.bfloat16)
    wsf = ws.astype(jnp.bfloat16)
    b1f = b1.reshape(1, cout)
    b2f = b2.reshape(1, cout)
    bsf = bs.reshape(cout, 1)

    xcm_spec = pl.BlockSpec((imgs, cin, HW), lambda n: (n, 0, 0))
    y1_spec = pl.BlockSpec((imgs, HW, cout), lambda n: (n, 0, 0))
    cm_spec = pl.BlockSpec((imgs, cout, HW), lambda n: (n, 0, 0))
    cmc_spec = pl.BlockSpec((imgs_c, cout, HW), lambda n: (n, 0, 0))
    st4_spec = pl.BlockSpec((1, 4, cout), lambda n: (n, 0, 0))
    st2_spec = pl.BlockSpec((1, 2, cout), lambda n: (n, 0, 0))

    y1, sct, sta = pl.pallas_call(
        functools.partial(_conv1_body, H=H, W=W, imgs=imgs),
        grid=(G,),
        in_specs=[xcm_spec, _whole(w1f.shape), _whole((1, cout)),
                  _whole(wsf.shape), _whole((cout, 1))],
        out_specs=(y1_spec, cm_spec, st4_spec),
        out_shape=(
            jax.ShapeDtypeStruct((N, HW, cout), jnp.bfloat16),
            jax.ShapeDtypeStruct((N, cout, HW), jnp.bfloat16),
            jax.ShapeDtypeStruct((G, 4, cout), jnp.float32),
        ),
        scratch_shapes=[pltpu.VMEM((H + 2, W + 2, cin), jnp.bfloat16)],
        compiler_params=_cparams(),
        cost_estimate=pl.CostEstimate(
            flops=2 * count * (9 * cin + cin) * cout, transcendentals=0,
            bytes_accessed=4 * count * cin + 2 * 2 * count * cout),
    )(xf, w1f, b1f, wsf, bsf)

    y2, stb = pl.pallas_call(
        functools.partial(_conv2_body, H=H, W=W, imgs=imgs, count=count),
        grid=(G,),
        in_specs=[y1_spec, _whole((G, 4, cout)), _whole(w2f.shape),
                  _whole((1, cout))],
        out_specs=(y1_spec, st2_spec),
        out_shape=(jax.ShapeDtypeStruct((N, HW, cout), jnp.bfloat16),
                   jax.ShapeDtypeStruct((G, 2, cout), jnp.float32)),
        scratch_shapes=[pltpu.VMEM((H + 2, W + 2, cout), jnp.bfloat16)],
        compiler_params=_cparams(),
        cost_estimate=pl.CostEstimate(
            flops=2 * count * 9 * cout * cout, transcendentals=0,
            bytes_accessed=2 * 2 * count * cout),
    )(y1, sta, w2f, b2f)

    y1c_spec = pl.BlockSpec((imgs_c, HW, cout), lambda n: (n, 0, 0))
    out = pl.pallas_call(
        functools.partial(_add_relu_body, count=count, imgs=imgs_c),
        grid=(Gc,),
        in_specs=[y1c_spec, cmc_spec, _whole((G, 4, cout)),
                  _whole((G, 2, cout))],
        out_specs=cmc_spec,
        out_shape=jax.ShapeDtypeStruct((N, cout, HW), jnp.float32),
        compiler_params=_cparams(),
        cost_estimate=pl.CostEstimate(
            flops=6 * count * cout, transcendentals=0,
            bytes_accessed=2 * 2 * count * cout + 4 * count * cout),
    )(y2, sct, sta, stb)

    return out.reshape(N, cout, H, W)


# speedup vs baseline: 1.7376x; 1.4057x over previous
import functools

import jax
import jax.numpy as jnp
from jax.experimental import pallas as pl
from jax.experimental.pallas import tpu as pltpu

_EPS = 1e-5
_VMEM_LIMIT = 64 * 1024 * 1024


def _cparams():
    return pltpu.CompilerParams(
        dimension_semantics=("parallel",),
        vmem_limit_bytes=_VMEM_LIMIT,
    )


def _whole(shape):
    shape = tuple(shape)
    return pl.BlockSpec(shape, lambda n: (0,) * len(shape))


def _im2col(xp, H, W):
    C = xp.shape[-1]
    return jnp.concatenate(
        [xp[dy:dy + H, dx:dx + W, :].reshape(H * W, C)
         for dy in range(3) for dx in range(3)], axis=-1)


def _scale_shift_rows(sum_row, sq_row, count):
    mean = sum_row / count
    var = jnp.maximum(sq_row / count - mean * mean, 0.0)
    inv = jax.lax.rsqrt(var + _EPS)
    return inv, -mean * inv


def _conv1_body(x_ref, w1_ref, b1_ref, ws_ref, bs_ref,
                y1_ref, sc_ref, st_ref, xp, *, H, W, imgs):
    cin = x_ref.shape[-1]
    cout = w1_ref.shape[-1]

    s1 = jnp.zeros((1, cout), jnp.float32)
    q1 = jnp.zeros((1, cout), jnp.float32)
    ss = jnp.zeros((1, cout), jnp.float32)
    qs = jnp.zeros((1, cout), jnp.float32)
    for i in range(imgs):
        xb = x_ref[i].astype(jnp.bfloat16)

        xp[...] = jnp.zeros((H + 2, W + 2, cin), jnp.bfloat16)
        xp[1:1 + H, 1:1 + W, :] = xb.reshape(H, W, cin)

        patches = _im2col(xp[...], H, W)
        y1 = jnp.dot(patches, w1_ref[...],
                     preferred_element_type=jnp.float32) + b1_ref[...]
        s1 = s1 + jnp.sum(y1, axis=0, keepdims=True)
        q1 = q1 + jnp.sum(y1 * y1, axis=0, keepdims=True)
        y1_ref[i] = y1.astype(jnp.bfloat16)

        sc = jnp.dot(xb, ws_ref[...],
                     preferred_element_type=jnp.float32) + bs_ref[...]
        ss = ss + jnp.sum(sc, axis=0, keepdims=True)
        qs = qs + jnp.sum(sc * sc, axis=0, keepdims=True)
        sc_ref[i] = sc.astype(jnp.bfloat16)

    st_ref[...] = jnp.concatenate([s1, q1, ss, qs], axis=0)[None]


def _conv2_body(y1_ref, sta_ref, w2_ref, b2_ref, y2_ref, st_ref, hp,
                *, H, W, imgs, count):
    cout = w2_ref.shape[-1]
    tot = jnp.sum(sta_ref[...], axis=0)
    scale, shift = _scale_shift_rows(tot[0:1], tot[1:2], count)

    s2 = jnp.zeros((1, cout), jnp.float32)
    q2 = jnp.zeros((1, cout), jnp.float32)
    for i in range(imgs):
        h1 = jnp.maximum(y1_ref[i].astype(jnp.float32) * scale + shift, 0.0)

        hp[...] = jnp.zeros((H + 2, W + 2, cout), jnp.bfloat16)
        hp[1:1 + H, 1:1 + W, :] = h1.astype(jnp.bfloat16).reshape(H, W, cout)

        patches = _im2col(hp[...], H, W)
        y2 = jnp.dot(patches, w2_ref[...],
                     preferred_element_type=jnp.float32) + b2_ref[...]
        s2 = s2 + jnp.sum(y2, axis=0, keepdims=True)
        q2 = q2 + jnp.sum(y2 * y2, axis=0, keepdims=True)
        y2_ref[i] = y2.astype(jnp.bfloat16)

    st_ref[...] = jnp.concatenate([s2, q2], axis=0)[None]


def _add_relu_body(y2_ref, sc_ref, sta_ref, stb_ref, o_ref, *, count):
    tota = jnp.sum(sta_ref[...], axis=0)
    totb = jnp.sum(stb_ref[...], axis=0)
    s2, t2 = _scale_shift_rows(totb[0:1], totb[1:2], count)
    ss, ts = _scale_shift_rows(tota[2:3], tota[3:4], count)
    y2 = y2_ref[...].astype(jnp.float32) * s2 + t2
    sc = sc_ref[...].astype(jnp.float32) * ss + ts
    o_ref[...] = jnp.maximum(y2 + sc, 0.0)


def kernel(x, w1, b1, w2, b2, ws, bs):
    N, cin, H, W = x.shape
    cout = w1.shape[-1]
    HW = H * W
    count = N * HW
    imgs = 8 if N % 8 == 0 else (2 if N % 2 == 0 else 1)
    imgs_c = 4 if N % 4 == 0 else 1
    G = N // imgs
    Gc = N // imgs_c

    xh = jnp.transpose(x, (0, 2, 3, 1)).astype(jnp.float32).reshape(N, HW, cin)

    w1f = w1.reshape(9 * cin, cout).astype(jnp.bfloat16)
    w2f = w2.reshape(9 * cout, cout).astype(jnp.bfloat16)
    wsf = ws.astype(jnp.bfloat16)
    b1f = b1.reshape(1, cout)
    b2f = b2.reshape(1, cout)
    bsf = bs.reshape(1, cout)

    x_spec = pl.BlockSpec((imgs, HW, cin), lambda n: (n, 0, 0))
    row_spec = pl.BlockSpec((imgs, HW, cout), lambda n: (n, 0, 0))
    rowc_spec = pl.BlockSpec((imgs_c, HW, cout), lambda n: (n, 0, 0))
    st4_spec = pl.BlockSpec((1, 4, cout), lambda n: (n, 0, 0))
    st2_spec = pl.BlockSpec((1, 2, cout), lambda n: (n, 0, 0))

    y1, sc, sta = pl.pallas_call(
        functools.partial(_conv1_body, H=H, W=W, imgs=imgs),
        grid=(G,),
        in_specs=[x_spec, _whole(w1f.shape), _whole((1, cout)),
                  _whole(wsf.shape), _whole((1, cout))],
        out_specs=(row_spec, row_spec, st4_spec),
        out_shape=(
            jax.ShapeDtypeStruct((N, HW, cout), jnp.bfloat16),
            jax.ShapeDtypeStruct((N, HW, cout), jnp.bfloat16),
            jax.ShapeDtypeStruct((G, 4, cout), jnp.float32),
        ),
        scratch_shapes=[pltpu.VMEM((H + 2, W + 2, cin), jnp.bfloat16)],
        compiler_params=_cparams(),
        cost_estimate=pl.CostEstimate(
            flops=2 * count * (9 * cin + cin) * cout, transcendentals=0,
            bytes_accessed=4 * count * cin + 2 * 2 * count * cout),
    )(xh, w1f, b1f, wsf, bsf)

    y2, stb = pl.pallas_call(
        functools.partial(_conv2_body, H=H, W=W, imgs=imgs, count=count),
        grid=(G,),
        in_specs=[row_spec, _whole((G, 4, cout)), _whole(w2f.shape),
                  _whole((1, cout))],
        out_specs=(row_spec, st2_spec),
        out_shape=(jax.ShapeDtypeStruct((N, HW, cout), jnp.bfloat16),
                   jax.ShapeDtypeStruct((G, 2, cout), jnp.float32)),
        scratch_shapes=[pltpu.VMEM((H + 2, W + 2, cout), jnp.bfloat16)],
        compiler_params=_cparams(),
        cost_estimate=pl.CostEstimate(
            flops=2 * count * 9 * cout * cout, transcendentals=0,
            bytes_accessed=2 * 2 * count * cout),
    )(y1, sta, w2f, b2f)

    out = pl.pallas_call(
        functools.partial(_add_relu_body, count=count),
        grid=(Gc,),
        in_specs=[rowc_spec, rowc_spec, _whole((G, 4, cout)),
                  _whole((G, 2, cout))],
        out_specs=rowc_spec,
        out_shape=jax.ShapeDtypeStruct((N, HW, cout), jnp.float32),
        compiler_params=_cparams(),
        cost_estimate=pl.CostEstimate(
            flops=6 * count * cout, transcendentals=0,
            bytes_accessed=2 * 2 * count * cout + 4 * count * cout),
    )(y2, sc, sta, stb)

    return jnp.transpose(out.reshape(N, H, W, cout), (0, 3, 1, 2))


# speedup vs baseline: 1.7547x; 1.0099x over previous
import functools

import jax
import jax.numpy as jnp
from jax.experimental import pallas as pl
from jax.experimental.pallas import tpu as pltpu

_EPS = 1e-5
_VMEM_LIMIT = 64 * 1024 * 1024


def _cparams():
    return pltpu.CompilerParams(
        dimension_semantics=("parallel",),
        vmem_limit_bytes=_VMEM_LIMIT,
    )


def _whole(shape):
    shape = tuple(shape)
    return pl.BlockSpec(shape, lambda n: (0,) * len(shape))


def _im2col(xp, H, W):
    C = xp.shape[-1]
    return jnp.concatenate(
        [xp[dy:dy + H, dx:dx + W, :].reshape(H * W, C)
         for dy in range(3) for dx in range(3)], axis=-1)


def _scale_shift_rows(sum_row, sq_row, count):
    mean = sum_row / count
    var = jnp.maximum(sq_row / count - mean * mean, 0.0)
    inv = jax.lax.rsqrt(var + _EPS)
    return inv, -mean * inv


def _conv1_body(x_ref, w1_ref, b1_ref, y1_ref, st_ref, g_ref, xp,
                *, H, W, imgs):
    cin = x_ref.shape[-1]
    cout = w1_ref.shape[-1]

    s1 = jnp.zeros((1, cout), jnp.float32)
    q1 = jnp.zeros((1, cout), jnp.float32)
    xs = jnp.zeros((1, cin), jnp.float32)
    gacc = jnp.zeros((cin, cin), jnp.float32)
    for i in range(imgs):
        xb = x_ref[i].astype(jnp.bfloat16)

        xp[...] = jnp.zeros((H + 2, W + 2, cin), jnp.bfloat16)
        xp[1:1 + H, 1:1 + W, :] = xb.reshape(H, W, cin)

        patches = _im2col(xp[...], H, W)
        y1 = jnp.dot(patches, w1_ref[...],
                     preferred_element_type=jnp.float32) + b1_ref[...]
        s1 = s1 + jnp.sum(y1, axis=0, keepdims=True)
        q1 = q1 + jnp.sum(y1 * y1, axis=0, keepdims=True)
        y1_ref[i] = y1.astype(jnp.bfloat16)

        xs = xs + jnp.sum(x_ref[i], axis=0, keepdims=True)
        gacc = gacc + jax.lax.dot_general(
            xb, xb, (((0,), (0,)), ((), ())),
            preferred_element_type=jnp.float32)

    if cout > cin:
        xs = jnp.concatenate(
            [xs, jnp.zeros((1, cout - cin), jnp.float32)], axis=1)
    else:
        xs = xs[:, :cout]
    st_ref[...] = jnp.concatenate([s1, q1, xs], axis=0)[None]
    g_ref[...] = gacc[None]


def _conv2_body(y1_ref, sta_ref, w2_ref, b2_ref, y2_ref, st_ref, hp,
                *, H, W, imgs, count):
    cout = w2_ref.shape[-1]
    tot = jnp.sum(sta_ref[...], axis=0)
    scale, shift = _scale_shift_rows(tot[0:1], tot[1:2], count)

    s2 = jnp.zeros((1, cout), jnp.float32)
    q2 = jnp.zeros((1, cout), jnp.float32)
    for i in range(imgs):
        h1 = jnp.maximum(y1_ref[i].astype(jnp.float32) * scale + shift, 0.0)

        hp[...] = jnp.zeros((H + 2, W + 2, cout), jnp.bfloat16)
        hp[1:1 + H, 1:1 + W, :] = h1.astype(jnp.bfloat16).reshape(H, W, cout)

        patches = _im2col(hp[...], H, W)
        y2 = jnp.dot(patches, w2_ref[...],
                     preferred_element_type=jnp.float32) + b2_ref[...]
        s2 = s2 + jnp.sum(y2, axis=0, keepdims=True)
        q2 = q2 + jnp.sum(y2 * y2, axis=0, keepdims=True)
        y2_ref[i] = y2.astype(jnp.bfloat16)

    st_ref[...] = jnp.concatenate([s2, q2], axis=0)[None]


def _add_relu_body(y2_ref, x_ref, ws_ref, stb_ref, bn_ref, o_ref,
                   *, imgs, count):
    totb = jnp.sum(stb_ref[...], axis=0)
    s2, t2 = _scale_shift_rows(totb[0:1], totb[1:2], count)
    bn = bn_ref[...]
    ss = bn[0:1]
    ts = bn[1:2]
    bs = bn[2:3]
    for i in range(imgs):
        sc = jnp.dot(x_ref[i].astype(jnp.bfloat16), ws_ref[...],
                     preferred_element_type=jnp.float32) + bs
        y2 = y2_ref[i].astype(jnp.float32) * s2 + t2
        o_ref[i] = jnp.maximum(y2 + sc * ss + ts, 0.0)


def kernel(x, w1, b1, w2, b2, ws, bs):
    N, cin, H, W = x.shape
    cout = w1.shape[-1]
    HW = H * W
    count = N * HW
    imgs = 8 if N % 8 == 0 else (2 if N % 2 == 0 else 1)
    imgs_c = 4 if N % 4 == 0 else 1
    G = N // imgs
    Gc = N // imgs_c

    xh = jnp.transpose(x, (0, 2, 3, 1)).astype(jnp.float32).reshape(N, HW, cin)

    w1f = w1.reshape(9 * cin, cout).astype(jnp.bfloat16)
    w2f = w2.reshape(9 * cout, cout).astype(jnp.bfloat16)
    wsf = ws.astype(jnp.bfloat16)
    b1f = b1.reshape(1, cout)
    b2f = b2.reshape(1, cout)

    x_spec = pl.BlockSpec((imgs, HW, cin), lambda n: (n, 0, 0))
    xc_spec = pl.BlockSpec((imgs_c, HW, cin), lambda n: (n, 0, 0))
    row_spec = pl.BlockSpec((imgs, HW, cout), lambda n: (n, 0, 0))
    rowc_spec = pl.BlockSpec((imgs_c, HW, cout), lambda n: (n, 0, 0))
    st3_spec = pl.BlockSpec((1, 3, cout), lambda n: (n, 0, 0))
    st2_spec = pl.BlockSpec((1, 2, cout), lambda n: (n, 0, 0))
    g_spec = pl.BlockSpec((1, cin, cin), lambda n: (n, 0, 0))

    y1, sta, gram = pl.pallas_call(
        functools.partial(_conv1_body, H=H, W=W, imgs=imgs),
        grid=(G,),
        in_specs=[x_spec, _whole(w1f.shape), _whole((1, cout))],
        out_specs=(row_spec, st3_spec, g_spec),
        out_shape=(
            jax.ShapeDtypeStruct((N, HW, cout), jnp.bfloat16),
            jax.ShapeDtypeStruct((G, 3, cout), jnp.float32),
            jax.ShapeDtypeStruct((G, cin, cin), jnp.float32),
        ),
        scratch_shapes=[pltpu.VMEM((H + 2, W + 2, cin), jnp.bfloat16)],
        compiler_params=_cparams(),
        cost_estimate=pl.CostEstimate(
            flops=2 * count * (9 * cin + cin) * cout, transcendentals=0,
            bytes_accessed=4 * count * cin + 2 * count * cout),
    )(xh, w1f, b1f)

    y2, stb = pl.pallas_call(
        functools.partial(_conv2_body, H=H, W=W, imgs=imgs, count=count),
        grid=(G,),
        in_specs=[row_spec, _whole((G, 3, cout)), _whole(w2f.shape),
                  _whole((1, cout))],
        out_specs=(row_spec, st2_spec),
        out_shape=(jax.ShapeDtypeStruct((N, HW, cout), jnp.bfloat16),
                   jax.ShapeDtypeStruct((G, 2, cout), jnp.float32)),
        scratch_shapes=[pltpu.VMEM((H + 2, W + 2, cout), jnp.bfloat16)],
        compiler_params=_cparams(),
        cost_estimate=pl.CostEstimate(
            flops=2 * count * 9 * cout * cout, transcendentals=0,
            bytes_accessed=2 * 2 * count * cout),
    )(y1, sta, w2f, b2f)

    tota = jnp.sum(sta, axis=0)
    xs = tota[2][:cin]
    Gm = jnp.sum(gram, axis=0)
    wsf32 = wsf.astype(jnp.float32)
    mproj = xs @ wsf32
    ssum = mproj + count * bs
    qsum = (jnp.sum(wsf32 * (Gm @ wsf32), axis=0)
            + 2.0 * bs * mproj + count * bs * bs)
    ss_, ts_ = _scale_shift_rows(ssum[None], qsum[None], count)
    bnrows = jnp.concatenate([ss_, ts_, bs[None]], axis=0)

    out = pl.pallas_call(
        functools.partial(_add_relu_body, imgs=imgs_c, count=count),
        grid=(Gc,),
        in_specs=[rowc_spec, xc_spec, _whole(wsf.shape),
                  _whole((G, 2, cout)), _whole((3, cout))],
        out_specs=rowc_spec,
        out_shape=jax.ShapeDtypeStruct((N, HW, cout), jnp.float32),
        compiler_params=_cparams(),
        cost_estimate=pl.CostEstimate(
            flops=2 * count * cin * cout + 6 * count * cout, transcendentals=0,
            bytes_accessed=2 * count * cout + 4 * count * cin
                           + 4 * count * cout),
    )(y2, xh, wsf, stb, bnrows)

    return jnp.transpose(out.reshape(N, H, W, cout), (0, 3, 1, 2))


# speedup vs baseline: 1.7857x; 1.0176x over previous
import functools

import jax
import jax.numpy as jnp
from jax.experimental import pallas as pl
from jax.experimental.pallas import tpu as pltpu

_EPS = 1e-5
_VMEM_LIMIT = 64 * 1024 * 1024


def _cparams():
    return pltpu.CompilerParams(
        dimension_semantics=("parallel",),
        vmem_limit_bytes=_VMEM_LIMIT,
    )


def _whole(shape):
    shape = tuple(shape)
    return pl.BlockSpec(shape, lambda n: (0,) * len(shape))


def _im2col(xp, H, W):
    C = xp.shape[-1]
    return jnp.concatenate(
        [xp[dy:dy + H, dx:dx + W, :].reshape(H * W, C)
         for dy in range(3) for dx in range(3)], axis=-1)


def _scale_shift_rows(sum_row, sq_row, count):
    mean = sum_row / count
    var = jnp.maximum(sq_row / count - mean * mean, 0.0)
    inv = jax.lax.rsqrt(var + _EPS)
    return inv, -mean * inv


def _conv1_body(x_ref, w1_ref, b1_ref, y1_ref, st_ref, g_ref, xp,
                *, H, W, imgs):
    cin = x_ref.shape[-1]
    cout = w1_ref.shape[-1]

    s1 = jnp.zeros((1, cout), jnp.float32)
    q1 = jnp.zeros((1, cout), jnp.float32)
    xs = jnp.zeros((1, cin), jnp.float32)
    gacc = jnp.zeros((cin, cin), jnp.float32)
    xp[...] = jnp.zeros((H + 2, W + 2, cin), jnp.bfloat16)
    for i in range(imgs):
        xb = x_ref[i].astype(jnp.bfloat16)
        xp[1:1 + H, 1:1 + W, :] = xb.reshape(H, W, cin)

        patches = _im2col(xp[...], H, W)
        y1 = jnp.dot(patches, w1_ref[...],
                     preferred_element_type=jnp.float32) + b1_ref[...]
        s1 = s1 + jnp.sum(y1, axis=0, keepdims=True)
        q1 = q1 + jnp.sum(y1 * y1, axis=0, keepdims=True)
        y1_ref[i] = y1.astype(jnp.bfloat16)

        xs = xs + jnp.sum(x_ref[i], axis=0, keepdims=True)
        gacc = gacc + jax.lax.dot_general(
            xb, xb, (((0,), (0,)), ((), ())),
            preferred_element_type=jnp.float32)

    if cout > cin:
        xs = jnp.concatenate(
            [xs, jnp.zeros((1, cout - cin), jnp.float32)], axis=1)
    else:
        xs = xs[:, :cout]
    st_ref[...] = jnp.concatenate([s1, q1, xs], axis=0)[None]
    g_ref[...] = gacc[None]


def _conv2_body(y1_ref, sta_ref, w2_ref, b2_ref, y2_ref, st_ref, hp,
                *, H, W, imgs, count):
    cout = w2_ref.shape[-1]
    tot = jnp.sum(sta_ref[...], axis=0)
    scale, shift = _scale_shift_rows(tot[0:1], tot[1:2], count)

    s2 = jnp.zeros((1, cout), jnp.float32)
    q2 = jnp.zeros((1, cout), jnp.float32)
    hp[...] = jnp.zeros((H + 2, W + 2, cout), jnp.bfloat16)
    for i in range(imgs):
        h1 = jnp.maximum(y1_ref[i].astype(jnp.float32) * scale + shift, 0.0)
        hp[1:1 + H, 1:1 + W, :] = h1.astype(jnp.bfloat16).reshape(H, W, cout)

        patches = _im2col(hp[...], H, W)
        y2 = jnp.dot(patches, w2_ref[...],
                     preferred_element_type=jnp.float32) + b2_ref[...]
        s2 = s2 + jnp.sum(y2, axis=0, keepdims=True)
        q2 = q2 + jnp.sum(y2 * y2, axis=0, keepdims=True)
        y2_ref[i] = y2.astype(jnp.bfloat16)

    st_ref[...] = jnp.concatenate([s2, q2], axis=0)[None]


def _add_relu_body(y2_ref, x_ref, ws_ref, stb_ref, bn_ref, o_ref,
                   *, imgs, count):
    totb = jnp.sum(stb_ref[...], axis=0)
    s2, t2 = _scale_shift_rows(totb[0:1], totb[1:2], count)
    bn = bn_ref[...]
    ss = bn[0:1]
    ts = bn[1:2]
    bs = bn[2:3]
    for i in range(imgs):
        sc = jnp.dot(x_ref[i].astype(jnp.bfloat16), ws_ref[...],
                     preferred_element_type=jnp.float32) + bs
        y2 = y2_ref[i].astype(jnp.float32) * s2 + t2
        o_ref[i] = jnp.maximum(y2 + sc * ss + ts, 0.0)


def kernel(x, w1, b1, w2, b2, ws, bs):
    N, cin, H, W = x.shape
    cout = w1.shape[-1]
    HW = H * W
    count = N * HW
    imgs = 8 if N % 8 == 0 else (2 if N % 2 == 0 else 1)
    imgs_c = 4 if N % 4 == 0 else 1
    G = N // imgs
    Gc = N // imgs_c

    xh = jnp.transpose(x, (0, 2, 3, 1)).astype(jnp.float32).reshape(N, HW, cin)

    w1f = w1.reshape(9 * cin, cout).astype(jnp.bfloat16)
    w2f = w2.reshape(9 * cout, cout).astype(jnp.bfloat16)
    wsf = ws.astype(jnp.bfloat16)
    b1f = b1.reshape(1, cout)
    b2f = b2.reshape(1, cout)

    x_spec = pl.BlockSpec((imgs, HW, cin), lambda n: (n, 0, 0))
    xc_spec = pl.BlockSpec((imgs_c, HW, cin), lambda n: (n, 0, 0))
    row_spec = pl.BlockSpec((imgs, HW, cout), lambda n: (n, 0, 0))
    rowc_spec = pl.BlockSpec((imgs_c, HW, cout), lambda n: (n, 0, 0))
    st3_spec = pl.BlockSpec((1, 3, cout), lambda n: (n, 0, 0))
    st2_spec = pl.BlockSpec((1, 2, cout), lambda n: (n, 0, 0))
    g_spec = pl.BlockSpec((1, cin, cin), lambda n: (n, 0, 0))

    y1, sta, gram = pl.pallas_call(
        functools.partial(_conv1_body, H=H, W=W, imgs=imgs),
        grid=(G,),
        in_specs=[x_spec, _whole(w1f.shape), _whole((1, cout))],
        out_specs=(row_spec, st3_spec, g_spec),
        out_shape=(
            jax.ShapeDtypeStruct((N, HW, cout), jnp.bfloat16),
            jax.ShapeDtypeStruct((G, 3, cout), jnp.float32),
            jax.ShapeDtypeStruct((G, cin, cin), jnp.float32),
        ),
        scratch_shapes=[pltpu.VMEM((H + 2, W + 2, cin), jnp.bfloat16)],
        compiler_params=_cparams(),
        cost_estimate=pl.CostEstimate(
            flops=2 * count * (9 * cin + cin) * cout, transcendentals=0,
            bytes_accessed=4 * count * cin + 2 * count * cout),
    )(xh, w1f, b1f)

    y2, stb = pl.pallas_call(
        functools.partial(_conv2_body, H=H, W=W, imgs=imgs, count=count),
        grid=(G,),
        in_specs=[row_spec, _whole((G, 3, cout)), _whole(w2f.shape),
                  _whole((1, cout))],
        out_specs=(row_spec, st2_spec),
        out_shape=(jax.ShapeDtypeStruct((N, HW, cout), jnp.bfloat16),
                   jax.ShapeDtypeStruct((G, 2, cout), jnp.float32)),
        scratch_shapes=[pltpu.VMEM((H + 2, W + 2, cout), jnp.bfloat16)],
        compiler_params=_cparams(),
        cost_estimate=pl.CostEstimate(
            flops=2 * count * 9 * cout * cout, transcendentals=0,
            bytes_accessed=2 * 2 * count * cout),
    )(y1, sta, w2f, b2f)

    tota = jnp.sum(sta, axis=0)
    xs = tota[2][:cin]
    Gm = jnp.sum(gram, axis=0)
    wsf32 = wsf.astype(jnp.float32)
    mproj = xs @ wsf32
    ssum = mproj + count * bs
    qsum = (jnp.sum(wsf32 * (Gm @ wsf32), axis=0)
            + 2.0 * bs * mproj + count * bs * bs)
    ss_, ts_ = _scale_shift_rows(ssum[None], qsum[None], count)
    bnrows = jnp.concatenate([ss_, ts_, bs[None]], axis=0)

    out = pl.pallas_call(
        functools.partial(_add_relu_body, imgs=imgs_c, count=count),
        grid=(Gc,),
        in_specs=[rowc_spec, xc_spec, _whole(wsf.shape),
                  _whole((G, 2, cout)), _whole((3, cout))],
        out_specs=rowc_spec,
        out_shape=jax.ShapeDtypeStruct((N, HW, cout), jnp.float32),
        compiler_params=_cparams(),
        cost_estimate=pl.CostEstimate(
            flops=2 * count * cin * cout + 6 * count * cout, transcendentals=0,
            bytes_accessed=2 * count * cout + 4 * count * cin
                           + 4 * count * cout),
    )(y2, xh, wsf, stb, bnrows)

    return jnp.transpose(out.reshape(N, H, W, cout), (0, 3, 1, 2))
